# Initial kernel scaffold; baseline (speedup 1.0000x reference)
import jax
import jax.numpy as jnp
from jax.experimental import pallas as pl


def kernel(x, edge_index, batch, W1, b1, W2, b2, W3, b3, W4, b4):
    raise NotImplementedError("write your pallas kernel here")


# speedup vs baseline: 10.2204x; 10.2204x over previous
import functools

import jax
import jax.numpy as jnp
from jax import lax
from jax.experimental import pallas as pl
from jax.experimental.pallas import tpu as pltpu
from jax.experimental.pallas import tpu_sc as plsc

_N = 10000
_NP = 10240
_E = 320000
_G = 64
_NC = 2
_NS = 16
_NW = _NC * _NS
_EPW = _E // _NW
_K = 80
_NCHUNK = _EPW // _K
_RPT = _NP // _NS
_ZCH = 80
_DEGW = 16

_mesh = plsc.VectorSubcoreMesh(core_axis_name="c", subcore_axis_name="s")
_sc_params = pltpu.CompilerParams(use_tc_tiling_on_sc=False)


@functools.partial(
    pl.kernel,
    mesh=_mesh,
    compiler_params=_sc_params,
    out_type=jax.ShapeDtypeStruct((_NC, _NP, _DEGW), jnp.float32),
    scratch_types=[
        pltpu.VMEM((_K,), jnp.int32),
        pltpu.VMEM((_K, _DEGW), jnp.float32),
        pltpu.VMEM((_ZCH, _DEGW), jnp.float32),
        pltpu.VMEM_SHARED((_NP, _DEGW), jnp.float32),
    ],
)
def _deg(dst_hbm, out_hbm, idx_v, ones_v, zbuf_v, table_s):
    c = lax.axis_index("c")
    s = lax.axis_index("s")
    wid = s * _NC + c

    def _fill(i, _):
        ones_v[i, :] = jnp.ones((_DEGW,), jnp.float32)
        zbuf_v[i, :] = jnp.zeros((_DEGW,), jnp.float32)
        return 0

    lax.fori_loop(0, _K, _fill, 0)
    r0 = s * _RPT

    def _zero(z, _):
        pltpu.sync_copy(zbuf_v, table_s.at[pl.ds(r0 + z * _ZCH, _ZCH)])
        return 0

    lax.fori_loop(0, _RPT // _ZCH, _zero, 0)
    plsc.subcore_barrier()
    base = wid * _EPW

    def _step(t, _):
        pltpu.sync_copy(dst_hbm.at[pl.ds(base + t * _K, _K)], idx_v)
        pltpu.sync_copy(ones_v, table_s.at[idx_v], add=True)
        return 0

    lax.fori_loop(0, _NCHUNK, _step, 0)
    plsc.subcore_barrier()

    def _out(z, _):
        rr = r0 + z * _ZCH
        pltpu.sync_copy(table_s.at[pl.ds(rr, _ZCH)], out_hbm.at[c, pl.ds(rr, _ZCH)])
        return 0

    lax.fori_loop(0, _RPT // _ZCH, _out, 0)


def _make_scatter(feat):
    nv = feat // 16

    @functools.partial(
        pl.kernel,
        mesh=_mesh,
        compiler_params=_sc_params,
        out_type=jax.ShapeDtypeStruct((_NC, _NP, feat), jnp.float32),
        scratch_types=[
            pltpu.VMEM((_K,), jnp.int32),
            pltpu.VMEM((_K,), jnp.int32),
            pltpu.VMEM((_K, feat), jnp.float32),
            pltpu.VMEM((_ZCH, feat), jnp.float32),
            pltpu.VMEM_SHARED((_NP, feat), jnp.float32),
            pltpu.SemaphoreType.DMA,
        ],
    )
    def _scatter(y_hbm, src_hbm, dst_hbm, out_hbm, sidx, didx, rows, zbuf,
                 table, sem):
        c = lax.axis_index("c")
        s = lax.axis_index("s")
        wid = s * _NC + c

        def _zfill(i, _):
            for j in range(nv):
                zbuf[i, pl.ds(j * 16, 16)] = jnp.zeros((16,), jnp.float32)
            return 0

        lax.fori_loop(0, _ZCH, _zfill, 0)
        r0 = s * _RPT

        def _zero(z, _):
            pltpu.sync_copy(zbuf, table.at[pl.ds(r0 + z * _ZCH, _ZCH)])
            return 0

        lax.fori_loop(0, _RPT // _ZCH, _zero, 0)
        plsc.subcore_barrier()
        base = wid * _EPW

        def _step(t, _):
            pltpu.sync_copy(src_hbm.at[pl.ds(base + t * _K, _K)], sidx)
            pltpu.sync_copy(dst_hbm.at[pl.ds(base + t * _K, _K)], didx)
            pltpu.async_copy(y_hbm.at[sidx], rows, sem).wait()
            pltpu.sync_copy(rows, table.at[didx], add=True)
            return 0

        lax.fori_loop(0, _NCHUNK, _step, 0)
        plsc.subcore_barrier()

        def _out(z, _):
            rr = r0 + z * _ZCH
            pltpu.sync_copy(table.at[pl.ds(rr, _ZCH)],
                            out_hbm.at[c, pl.ds(rr, _ZCH)])
            return 0

        lax.fori_loop(0, _RPT // _ZCH, _out, 0)

    return _scatter


_scatter128 = _make_scatter(128)
_scatter64 = _make_scatter(64)


def _dinv_of(degt):
    def body(d_ref, o_ref):
        d = d_ref[0] + d_ref[1]
        o_ref[...] = lax.rsqrt(d[:, 0:1] + 1.0)

    return pl.pallas_call(
        body,
        out_shape=jax.ShapeDtypeStruct((_NP, 1), jnp.float32),
    )(degt)


def _matmul_scale(h, w, dinv, f_in, f_out):
    blk = 1024

    def body(h_ref, w_ref, d_ref, o_ref):
        o_ref[...] = d_ref[...] * jnp.dot(
            h_ref[...], w_ref[...], preferred_element_type=jnp.float32)

    return pl.pallas_call(
        body,
        grid=(_NP // blk,),
        in_specs=[
            pl.BlockSpec((blk, f_in), lambda i: (i, 0)),
            pl.BlockSpec((f_in, f_out), lambda i: (0, 0)),
            pl.BlockSpec((blk, 1), lambda i: (i, 0)),
        ],
        out_specs=pl.BlockSpec((blk, f_out), lambda i: (i, 0)),
        out_shape=jax.ShapeDtypeStruct((_NP, f_out), jnp.float32),
    )(h, w, dinv)


def _combine(agg, y, dinv, b, feat):
    blk = 1024

    def body(a_ref, y_ref, d_ref, b_ref, o_ref):
        t = a_ref[0] + a_ref[1] + y_ref[...]
        o_ref[...] = jnp.maximum(d_ref[...] * t + b_ref[...], 0.0)

    return pl.pallas_call(
        body,
        grid=(_NP // blk,),
        in_specs=[
            pl.BlockSpec((_NC, blk, feat), lambda i: (0, i, 0)),
            pl.BlockSpec((blk, feat), lambda i: (i, 0)),
            pl.BlockSpec((blk, 1), lambda i: (i, 0)),
            pl.BlockSpec((1, feat), lambda i: (0, 0)),
        ],
        out_specs=pl.BlockSpec((blk, feat), lambda i: (i, 0)),
        out_shape=jax.ShapeDtypeStruct((_NP, feat), jnp.float32),
    )(agg, y, dinv, b)


def _pool(h, batchp, feat):
    def body(h_ref, b_ref, o_ref):
        hv = h_ref[...]
        bv = b_ref[...]

        def g_step(g, _):
            m = bv == g
            col = jnp.where(m, hv, -jnp.inf)
            o_ref[pl.ds(g, 1), :] = jnp.max(col, axis=0, keepdims=True)
            return 0

        lax.fori_loop(0, _G, g_step, 0)

    return pl.pallas_call(
        body,
        out_shape=jax.ShapeDtypeStruct((_G, feat), jnp.float32),
    )(h, batchp)


def kernel(x, edge_index, batch, W1, b1, W2, b2, W3, b3, W4, b4):
    src = edge_index[0]
    dst = edge_index[1]
    xp = jnp.pad(x.astype(jnp.float32), ((0, _NP - _N), (0, 0)))
    batchp = jnp.pad(batch, (0, _NP - _N), constant_values=-1).reshape(_NP, 1)

    degt = _deg(dst)
    dinv = _dinv_of(degt)

    h = xp
    dims = ((W1, b1, 128, 128), (W2, b2, 128, 64),
            (W3, b3, 64, 64), (W4, b4, 64, 64))
    for W, b, f_in, f_out in dims:
        y = _matmul_scale(h, W, dinv, f_in, f_out)
        scat = _scatter128 if f_out == 128 else _scatter64
        agg = scat(y, src, dst)
        h = _combine(agg, y, dinv, b.reshape(1, f_out), f_out)

    return _pool(h, batchp, 64)


# speedup vs baseline: 13.4694x; 1.3179x over previous
import functools

import jax
import jax.numpy as jnp
from jax import lax
from jax.experimental import pallas as pl
from jax.experimental.pallas import tpu as pltpu
from jax.experimental.pallas import tpu_sc as plsc

_N = 10000
_NP = 10240
_E = 320000
_G = 64
_NC = 2
_NS = 16
_NW = _NC * _NS
_K = 128
_NCHB = 79
_EP = _NW * _NCHB * _K
_RPT = _NP // _NS
_ZCH = 80
_DEGW = 16

_mesh = plsc.VectorSubcoreMesh(core_axis_name="c", subcore_axis_name="s")
_sc_params = pltpu.CompilerParams(use_tc_tiling_on_sc=False)


@functools.partial(
    pl.kernel,
    mesh=_mesh,
    compiler_params=_sc_params,
    out_type=jax.ShapeDtypeStruct((_NC, _NP, _DEGW), jnp.float32),
    scratch_types=[
        pltpu.VMEM((_NCHB, _K), jnp.int32),
        pltpu.VMEM((_K, _DEGW), jnp.float32),
        pltpu.VMEM((_ZCH, _DEGW), jnp.float32),
        pltpu.VMEM_SHARED((_NP, _DEGW), jnp.float32),
    ],
)
def _deg(dst_hbm, out_hbm, didx, ones_v, zbuf_v, table_s):
    c = lax.axis_index("c")
    s = lax.axis_index("s")
    wid = s * _NC + c
    pltpu.sync_copy(dst_hbm.at[wid], didx)

    def _fill(i, _):
        ones_v[i, :] = jnp.ones((_DEGW,), jnp.float32)
        return 0

    lax.fori_loop(0, _K, _fill, 0)

    def _zfill(i, _):
        zbuf_v[i, :] = jnp.zeros((_DEGW,), jnp.float32)
        return 0

    lax.fori_loop(0, _ZCH, _zfill, 0)
    r0 = s * _RPT

    def _zero(z, _):
        pltpu.sync_copy(zbuf_v, table_s.at[pl.ds(r0 + z * _ZCH, _ZCH)])
        return 0

    lax.fori_loop(0, _RPT // _ZCH, _zero, 0)
    plsc.subcore_barrier()

    def _step(t, _):
        pltpu.sync_copy(ones_v, table_s.at[didx.at[t]], add=True)
        return 0

    lax.fori_loop(0, _NCHB, _step, 0)
    plsc.subcore_barrier()

    def _out(z, _):
        rr = r0 + z * _ZCH
        pltpu.sync_copy(table_s.at[pl.ds(rr, _ZCH)], out_hbm.at[c, pl.ds(rr, _ZCH)])
        return 0

    lax.fori_loop(0, _RPT // _ZCH, _out, 0)


def _make_scatter(feat, k, nchb):
    nv = feat // 16

    @functools.partial(
        pl.kernel,
        mesh=_mesh,
        compiler_params=_sc_params,
        out_type=jax.ShapeDtypeStruct((_NC, _NP, feat), jnp.float32),
        scratch_types=[
            pltpu.VMEM((nchb, k), jnp.int32),
            pltpu.VMEM((nchb, k), jnp.int32),
            pltpu.VMEM((k, feat), jnp.float32),
            pltpu.VMEM((k, feat), jnp.float32),
            pltpu.VMEM_SHARED((_NP, feat), jnp.float32),
            pltpu.SemaphoreType.DMA,
            pltpu.SemaphoreType.DMA,
        ],
    )
    def _scatter(y_hbm, src_hbm, dst_hbm, out_hbm, sidx, didx, rows0, rows1,
                 table, sem0, sem1):
        c = lax.axis_index("c")
        s = lax.axis_index("s")
        wid = s * _NC + c
        pltpu.sync_copy(src_hbm.at[wid], sidx)
        pltpu.sync_copy(dst_hbm.at[wid], didx)

        def _zfill(i, _):
            for j in range(nv):
                rows0[i, pl.ds(j * 16, 16)] = jnp.zeros((16,), jnp.float32)
            return 0

        lax.fori_loop(0, k, _zfill, 0)
        r0 = s * _RPT

        def _zero(z, _):
            pltpu.sync_copy(rows0, table.at[pl.ds(r0 + z * k, k)])
            return 0

        lax.fori_loop(0, _RPT // k, _zero, 0)
        plsc.subcore_barrier()

        pltpu.async_copy(y_hbm.at[sidx.at[0]], rows0, sem0).wait()
        npair = (nchb - 1) // 2

        def _pair(p, _):
            t0 = 2 * p + 1
            cp1 = pltpu.async_copy(y_hbm.at[sidx.at[t0]], rows1, sem1)
            pltpu.sync_copy(rows0, table.at[didx.at[t0 - 1]], add=True)
            cp1.wait()
            cp0 = pltpu.async_copy(y_hbm.at[sidx.at[t0 + 1]], rows0, sem0)
            pltpu.sync_copy(rows1, table.at[didx.at[t0]], add=True)
            cp0.wait()
            return 0

        lax.fori_loop(0, npair, _pair, 0)
        pltpu.sync_copy(rows0, table.at[didx.at[2 * npair]], add=True)
        if nchb % 2 == 0:
            pltpu.async_copy(y_hbm.at[sidx.at[nchb - 1]], rows1, sem1).wait()
            pltpu.sync_copy(rows1, table.at[didx.at[nchb - 1]], add=True)
        plsc.subcore_barrier()

        def _out(z, _):
            rr = r0 + z * _ZCH
            pltpu.sync_copy(table.at[pl.ds(rr, _ZCH)],
                            out_hbm.at[c, pl.ds(rr, _ZCH)])
            return 0

        lax.fori_loop(0, _RPT // _ZCH, _out, 0)

    return _scatter


_scatter128 = _make_scatter(128, 64, 2 * _NCHB)
_scatter64 = _make_scatter(64, _K, _NCHB)


def _dinv_of(degt):
    def body(d_ref, o_ref):
        d = d_ref[0] + d_ref[1]
        o_ref[...] = lax.rsqrt(d[:, 0:1] + 1.0)

    return pl.pallas_call(
        body,
        out_shape=jax.ShapeDtypeStruct((_NP, 1), jnp.float32),
    )(degt)


def _matmul_scale(h, w, dinv, f_in, f_out):
    blk = 1024

    def body(h_ref, w_ref, d_ref, o_ref):
        o_ref[...] = d_ref[...] * jnp.dot(
            h_ref[...], w_ref[...], preferred_element_type=jnp.float32)

    return pl.pallas_call(
        body,
        grid=(_NP // blk,),
        in_specs=[
            pl.BlockSpec((blk, f_in), lambda i: (i, 0)),
            pl.BlockSpec((f_in, f_out), lambda i: (0, 0)),
            pl.BlockSpec((blk, 1), lambda i: (i, 0)),
        ],
        out_specs=pl.BlockSpec((blk, f_out), lambda i: (i, 0)),
        out_shape=jax.ShapeDtypeStruct((_NP, f_out), jnp.float32),
    )(h, w, dinv)


def _combine(agg, y, dinv, b, feat):
    blk = 1024

    def body(a_ref, y_ref, d_ref, b_ref, o_ref):
        t = a_ref[0] + a_ref[1] + y_ref[...]
        o_ref[...] = jnp.maximum(d_ref[...] * t + b_ref[...], 0.0)

    return pl.pallas_call(
        body,
        grid=(_NP // blk,),
        in_specs=[
            pl.BlockSpec((_NC, blk, feat), lambda i: (0, i, 0)),
            pl.BlockSpec((blk, feat), lambda i: (i, 0)),
            pl.BlockSpec((blk, 1), lambda i: (i, 0)),
            pl.BlockSpec((1, feat), lambda i: (0, 0)),
        ],
        out_specs=pl.BlockSpec((blk, feat), lambda i: (i, 0)),
        out_shape=jax.ShapeDtypeStruct((_NP, feat), jnp.float32),
    )(agg, y, dinv, b)


def _pool(h, batchp, feat):
    def body(h_ref, b_ref, o_ref):
        hv = h_ref[...]
        bv = b_ref[...]

        def g_step(g, _):
            m = bv == g
            col = jnp.where(m, hv, -jnp.inf)
            o_ref[pl.ds(g, 1), :] = jnp.max(col, axis=0, keepdims=True)
            return 0

        lax.fori_loop(0, _G, g_step, 0)

    return pl.pallas_call(
        body,
        out_shape=jax.ShapeDtypeStruct((_G, feat), jnp.float32),
    )(h, batchp)


def kernel(x, edge_index, batch, W1, b1, W2, b2, W3, b3, W4, b4):
    pad_e = _EP - _E
    src1 = jnp.concatenate([edge_index[0], jnp.zeros((pad_e,), jnp.int32)])
    dst1 = jnp.concatenate([edge_index[1],
                            jnp.full((pad_e,), _NP - 1, jnp.int32)])
    srcp = src1.reshape(_NW, _NCHB, _K)
    dstp = dst1.reshape(_NW, _NCHB, _K)
    srcp2 = src1.reshape(_NW, 2 * _NCHB, _K // 2)
    dstp2 = dst1.reshape(_NW, 2 * _NCHB, _K // 2)
    xp = jnp.pad(x.astype(jnp.float32), ((0, _NP - _N), (0, 0)))
    batchp = jnp.pad(batch, (0, _NP - _N), constant_values=-1).reshape(_NP, 1)

    degt = _deg(dstp)
    dinv = _dinv_of(degt)

    h = xp
    dims = ((W1, b1, 128, 128), (W2, b2, 128, 64),
            (W3, b3, 64, 64), (W4, b4, 64, 64))
    for W, b, f_in, f_out in dims:
        y = _matmul_scale(h, W, dinv, f_in, f_out)
        if f_out == 128:
            agg = _scatter128(y, srcp2, dstp2)
        else:
            agg = _scatter64(y, srcp, dstp)
        h = _combine(agg, y, dinv, b.reshape(1, f_out), f_out)

    return _pool(h, batchp, 64)


# speedup vs baseline: 15.0880x; 1.1202x over previous
import functools

import jax
import jax.numpy as jnp
from jax import lax
from jax.experimental import pallas as pl
from jax.experimental.pallas import tpu as pltpu
from jax.experimental.pallas import tpu_sc as plsc

_N = 10000
_NP = 10240
_E = 320000
_G = 64
_NC = 2
_NS = 16
_NW = _NC * _NS
_K = 128
_NCHB = 79
_EP = _NW * _NCHB * _K
_RPT = _NP // _NS
_ZCH = 80
_DEGW = 16

_mesh = plsc.VectorSubcoreMesh(core_axis_name="c", subcore_axis_name="s")
_sc_params = pltpu.CompilerParams(use_tc_tiling_on_sc=False)


@functools.partial(
    pl.kernel,
    mesh=_mesh,
    compiler_params=_sc_params,
    out_type=jax.ShapeDtypeStruct((_NC, _NP, _DEGW), jnp.float32),
    scratch_types=[
        pltpu.VMEM((_NCHB, _K), jnp.int32),
        pltpu.VMEM((_K, _DEGW), jnp.float32),
        pltpu.VMEM((_ZCH, _DEGW), jnp.float32),
        pltpu.VMEM_SHARED((_NP, _DEGW), jnp.float32),
        pltpu.SemaphoreType.DMA,
    ],
)
def _deg(dst_hbm, out_hbm, didx, ones_v, zbuf_v, table_s, sem):
    c = lax.axis_index("c")
    s = lax.axis_index("s")
    wid = s * _NC + c
    pltpu.sync_copy(dst_hbm.at[wid], didx)

    def _fill(i, _):
        ones_v[i, :] = jnp.ones((_DEGW,), jnp.float32)
        return 0

    lax.fori_loop(0, _K, _fill, 0)

    def _zfill(i, _):
        zbuf_v[i, :] = jnp.zeros((_DEGW,), jnp.float32)
        return 0

    lax.fori_loop(0, _ZCH, _zfill, 0)
    r0 = s * _RPT

    def _zero(z, _):
        pltpu.sync_copy(zbuf_v, table_s.at[pl.ds(r0 + z * _ZCH, _ZCH)])
        return 0

    lax.fori_loop(0, _RPT // _ZCH, _zero, 0)
    plsc.subcore_barrier()

    def _step(t, _):
        pltpu.async_copy(ones_v, table_s.at[didx.at[t]], sem, add=True)
        return 0

    lax.fori_loop(0, _NCHB, _step, 0)

    def _drain(t, _):
        pltpu.make_async_copy(ones_v, table_s.at[didx.at[0]], sem).wait()
        return 0

    lax.fori_loop(0, _NCHB, _drain, 0)
    plsc.subcore_barrier()

    def _out(z, _):
        rr = r0 + z * _ZCH
        pltpu.sync_copy(table_s.at[pl.ds(rr, _ZCH)], out_hbm.at[c, pl.ds(rr, _ZCH)])
        return 0

    lax.fori_loop(0, _RPT // _ZCH, _out, 0)


def _zero_stripe(rows0, table, feat, k, r0):
    nv = feat // 16

    def _zfill(i, _):
        for j in range(nv):
            rows0[i, pl.ds(j * 16, 16)] = jnp.zeros((16,), jnp.float32)
        return 0

    lax.fori_loop(0, k, _zfill, 0)

    def _zero(z, _):
        pltpu.sync_copy(rows0, table.at[pl.ds(r0 + z * k, k)])
        return 0

    lax.fori_loop(0, _RPT // k, _zero, 0)


def _ring_loop(yref, table, sidx, didx, rows, gsems, ssems, nchb, hdeep):
    nbuf = len(rows)

    def issue_gather(t, b):
        pltpu.async_copy(yref.at[sidx.at[t]], rows[b], gsems[b])

    def wait_gather(b):
        pltpu.make_async_copy(yref.at[sidx.at[0]], rows[b], gsems[b]).wait()

    def issue_scatter(t, b):
        pltpu.async_copy(rows[b], table.at[didx.at[t]], ssems[b], add=True)

    def wait_scatter(b):
        pltpu.make_async_copy(rows[b], table.at[didx.at[0]], ssems[b]).wait()

    for b in range(hdeep):
        issue_gather(b, b)
    nq = nchb // nbuf

    def _q(q, _):
        for b in range(nbuf):
            kk = nbuf * q + b
            bh = (b + hdeep) % nbuf

            @pl.when(kk >= hdeep)
            def _():
                wait_scatter(bh)

            @pl.when(kk + hdeep < nchb)
            def _():
                issue_gather(kk + hdeep, bh)

            wait_gather(b)
            issue_scatter(kk, b)
        return 0

    lax.fori_loop(0, nq, _q, 0)
    for kk in range(nbuf * nq, nchb):
        b = kk % nbuf
        bh = (b + hdeep) % nbuf
        if kk >= hdeep:
            wait_scatter(bh)
        if kk + hdeep < nchb:
            issue_gather(kk + hdeep, bh)
        wait_gather(b)
        issue_scatter(kk, b)
    for kk in range(max(nchb - hdeep, 0), nchb):
        wait_scatter(kk % nbuf)


def _make_scatter64(k, nchb, nbuf, hdeep):
    feat = 64

    @functools.partial(
        pl.kernel,
        mesh=_mesh,
        compiler_params=_sc_params,
        out_type=jax.ShapeDtypeStruct((_NC, _NP, feat), jnp.float32),
        scratch_types=(
            [pltpu.VMEM((nchb, k), jnp.int32),
             pltpu.VMEM((nchb, k), jnp.int32)]
            + [pltpu.VMEM((k, feat), jnp.float32) for _ in range(nbuf)]
            + [pltpu.VMEM_SHARED((_NP, feat), jnp.float32)]
            + [pltpu.SemaphoreType.DMA for _ in range(2 * nbuf)]
        ),
    )
    def _scatter(y_hbm, src_hbm, dst_hbm, out_hbm, *scr):
        sidx, didx = scr[0], scr[1]
        rows = scr[2:2 + nbuf]
        table = scr[2 + nbuf]
        gsems = scr[3 + nbuf:3 + 2 * nbuf]
        ssems = scr[3 + 2 * nbuf:3 + 3 * nbuf]
        c = lax.axis_index("c")
        s = lax.axis_index("s")
        wid = s * _NC + c
        pltpu.sync_copy(src_hbm.at[wid], sidx)
        pltpu.sync_copy(dst_hbm.at[wid], didx)
        r0 = s * _RPT
        _zero_stripe(rows[0], table, feat, k, r0)
        plsc.subcore_barrier()
        _ring_loop(y_hbm, table, sidx, didx, rows, gsems, ssems, nchb, hdeep)
        plsc.subcore_barrier()

        def _out(z, _):
            rr = r0 + z * _ZCH
            pltpu.sync_copy(table.at[pl.ds(rr, _ZCH)],
                            out_hbm.at[c, pl.ds(rr, _ZCH)])
            return 0

        lax.fori_loop(0, _RPT // _ZCH, _out, 0)

    return _scatter


def _make_scatter_fs(k, nchb, nbuf, hdeep):
    feat = 64

    @functools.partial(
        pl.kernel,
        mesh=_mesh,
        compiler_params=_sc_params,
        out_type=jax.ShapeDtypeStruct((_NC, _NP, feat), jnp.float32),
        scratch_types=(
            [pltpu.VMEM((nchb, k), jnp.int32),
             pltpu.VMEM((nchb, k), jnp.int32)]
            + [pltpu.VMEM((k, feat), jnp.float32) for _ in range(nbuf)]
            + [pltpu.VMEM_SHARED((_NP, feat), jnp.float32)]
            + [pltpu.SemaphoreType.DMA for _ in range(2 * nbuf)]
        ),
    )
    def _scatter(ya_hbm, yb_hbm, src_hbm, dst_hbm, out_hbm, *scr):
        sidx, didx = scr[0], scr[1]
        rows = scr[2:2 + nbuf]
        table = scr[2 + nbuf]
        gsems = scr[3 + nbuf:3 + 2 * nbuf]
        ssems = scr[3 + 2 * nbuf:3 + 3 * nbuf]
        c = lax.axis_index("c")
        s = lax.axis_index("s")
        pltpu.sync_copy(src_hbm.at[s], sidx)
        pltpu.sync_copy(dst_hbm.at[s], didx)
        r0 = s * _RPT
        _zero_stripe(rows[0], table, feat, k, r0)
        plsc.subcore_barrier()

        @pl.when(c == 0)
        def _():
            _ring_loop(ya_hbm, table, sidx, didx, rows, gsems, ssems, nchb,
                       hdeep)

        @pl.when(c == 1)
        def _():
            _ring_loop(yb_hbm, table, sidx, didx, rows, gsems, ssems, nchb,
                       hdeep)

        plsc.subcore_barrier()

        def _out(z, _):
            rr = r0 + z * _ZCH
            pltpu.sync_copy(table.at[pl.ds(rr, _ZCH)],
                            out_hbm.at[c, pl.ds(rr, _ZCH)])
            return 0

        lax.fori_loop(0, _RPT // _ZCH, _out, 0)

    return _scatter


_scatter128 = _make_scatter_fs(_K, 2 * _NCHB, 4, 2)
_scatter64 = _make_scatter64(_K, _NCHB, 6, 3)


def _dinv_of(degt):
    def body(d_ref, o_ref):
        d = d_ref[0] + d_ref[1]
        o_ref[...] = lax.rsqrt(d[:, 0:1] + 1.0)

    return pl.pallas_call(
        body,
        out_shape=jax.ShapeDtypeStruct((_NP, 1), jnp.float32),
    )(degt)


def _matmul_scale(h, w, dinv, f_in, f_out):
    blk = 1024

    def body(h_ref, w_ref, d_ref, o_ref):
        o_ref[...] = d_ref[...] * jnp.dot(
            h_ref[...], w_ref[...], preferred_element_type=jnp.float32)

    return pl.pallas_call(
        body,
        grid=(_NP // blk,),
        in_specs=[
            pl.BlockSpec((blk, f_in), lambda i: (i, 0)),
            pl.BlockSpec((f_in, f_out), lambda i: (0, 0)),
            pl.BlockSpec((blk, 1), lambda i: (i, 0)),
        ],
        out_specs=pl.BlockSpec((blk, f_out), lambda i: (i, 0)),
        out_shape=jax.ShapeDtypeStruct((_NP, f_out), jnp.float32),
    )(h, w, dinv)


def _matmul_scale2(h, w, dinv):
    blk = 1024

    def body(h_ref, w_ref, d_ref, oa_ref, ob_ref):
        y = d_ref[...] * jnp.dot(h_ref[...], w_ref[...],
                                 preferred_element_type=jnp.float32)
        oa_ref[...] = y[:, :64]
        ob_ref[...] = y[:, 64:]

    return pl.pallas_call(
        body,
        grid=(_NP // blk,),
        in_specs=[
            pl.BlockSpec((blk, 128), lambda i: (i, 0)),
            pl.BlockSpec((128, 128), lambda i: (0, 0)),
            pl.BlockSpec((blk, 1), lambda i: (i, 0)),
        ],
        out_specs=[pl.BlockSpec((blk, 64), lambda i: (i, 0)),
                   pl.BlockSpec((blk, 64), lambda i: (i, 0))],
        out_shape=[jax.ShapeDtypeStruct((_NP, 64), jnp.float32),
                   jax.ShapeDtypeStruct((_NP, 64), jnp.float32)],
    )(h, w, dinv)


def _combine_fs(agg, ya, yb, dinv, b):
    blk = 1024

    def body(a_ref, ya_ref, yb_ref, d_ref, b_ref, o_ref):
        left = a_ref[0] + ya_ref[...]
        right = a_ref[1] + yb_ref[...]
        t = jnp.concatenate([left, right], axis=1)
        o_ref[...] = jnp.maximum(d_ref[...] * t + b_ref[...], 0.0)

    return pl.pallas_call(
        body,
        grid=(_NP // blk,),
        in_specs=[
            pl.BlockSpec((_NC, blk, 64), lambda i: (0, i, 0)),
            pl.BlockSpec((blk, 64), lambda i: (i, 0)),
            pl.BlockSpec((blk, 64), lambda i: (i, 0)),
            pl.BlockSpec((blk, 1), lambda i: (i, 0)),
            pl.BlockSpec((1, 128), lambda i: (0, 0)),
        ],
        out_specs=pl.BlockSpec((blk, 128), lambda i: (i, 0)),
        out_shape=jax.ShapeDtypeStruct((_NP, 128), jnp.float32),
    )(agg, ya, yb, dinv, b)


def _combine(agg, y, dinv, b, feat):
    blk = 1024

    def body(a_ref, y_ref, d_ref, b_ref, o_ref):
        t = a_ref[0] + a_ref[1] + y_ref[...]
        o_ref[...] = jnp.maximum(d_ref[...] * t + b_ref[...], 0.0)

    return pl.pallas_call(
        body,
        grid=(_NP // blk,),
        in_specs=[
            pl.BlockSpec((_NC, blk, feat), lambda i: (0, i, 0)),
            pl.BlockSpec((blk, feat), lambda i: (i, 0)),
            pl.BlockSpec((blk, 1), lambda i: (i, 0)),
            pl.BlockSpec((1, feat), lambda i: (0, 0)),
        ],
        out_specs=pl.BlockSpec((blk, feat), lambda i: (i, 0)),
        out_shape=jax.ShapeDtypeStruct((_NP, feat), jnp.float32),
    )(agg, y, dinv, b)


def _pool(h, batchp, feat):
    def body(h_ref, b_ref, o_ref):
        hv = h_ref[...]
        bv = b_ref[...]

        def g_step(g, _):
            m = bv == g
            col = jnp.where(m, hv, -jnp.inf)
            o_ref[pl.ds(g, 1), :] = jnp.max(col, axis=0, keepdims=True)
            return 0

        lax.fori_loop(0, _G, g_step, 0)

    return pl.pallas_call(
        body,
        out_shape=jax.ShapeDtypeStruct((_G, feat), jnp.float32),
    )(h, batchp)


def kernel(x, edge_index, batch, W1, b1, W2, b2, W3, b3, W4, b4):
    pad_e = _EP - _E
    src1 = jnp.concatenate([edge_index[0], jnp.zeros((pad_e,), jnp.int32)])
    dst1 = jnp.concatenate([edge_index[1],
                            jnp.full((pad_e,), _NP - 1, jnp.int32)])
    srcp = src1.reshape(_NW, _NCHB, _K)
    dstp = dst1.reshape(_NW, _NCHB, _K)
    srcf = src1.reshape(_NS, 2 * _NCHB, _K)
    dstf = dst1.reshape(_NS, 2 * _NCHB, _K)
    xp = jnp.pad(x.astype(jnp.float32), ((0, _NP - _N), (0, 0)))
    batchp = jnp.pad(batch, (0, _NP - _N), constant_values=-1).reshape(_NP, 1)

    degt = _deg(dstp)
    dinv = _dinv_of(degt)

    ya, yb = _matmul_scale2(xp, W1, dinv)
    agg = _scatter128(ya, yb, srcf, dstf)
    h = _combine_fs(agg, ya, yb, dinv, b1.reshape(1, 128))

    for W, b, f_in in ((W2, b2, 128), (W3, b3, 64), (W4, b4, 64)):
        y = _matmul_scale(h, W, dinv, f_in, 64)
        agg = _scatter64(y, srcp, dstp)
        h = _combine(agg, y, dinv, b.reshape(1, 64), 64)

    return _pool(h, batchp, 64)
